# baseline (speedup 1.0000x reference)
import functools
import math

import jax
import jax.numpy as jnp
from jax import lax
from jax.experimental import pallas as pl
from jax.experimental.pallas import tpu as pltpu
from jax.experimental.pallas import tpu_sc as plsc

_D = 128
_H = 8
_DH = 16
_CUTOFF = 5.0
_NW = 32


def _sc_mesh():
    return plsc.VectorSubcoreMesh(core_axis_name="c", subcore_axis_name="s")


_SC_PARAMS = pltpu.CompilerParams(use_tc_tiling_on_sc=False)


def _sc_gather(table, idx, chunk):
    e = idx.shape[0]
    dt = table.shape[1]
    epw = e // _NW
    nchunks = epw // chunk

    @functools.partial(
        pl.kernel,
        out_type=jax.ShapeDtypeStruct((e, dt), table.dtype),
        mesh=_sc_mesh(),
        scratch_types=[
            pltpu.VMEM((chunk,), jnp.int32),
            pltpu.VMEM((chunk,), jnp.int32),
            pltpu.VMEM((chunk, dt), jnp.float32),
            pltpu.VMEM((chunk, dt), jnp.float32),
            pltpu.SemaphoreType.DMA,
            pltpu.SemaphoreType.DMA,
        ],
        compiler_params=_SC_PARAMS,
    )
    def k(tab_hbm, idx_hbm, out_hbm, i0, i1, r0, r1, s0, s1):
        cid = lax.axis_index("c")
        sid = lax.axis_index("s")
        base = (sid * 2 + cid) * epw
        bufs = ((i0, r0, s0), (i1, r1, s1))

        for b in range(2):
            iv, rv, sm = bufs[b]
            pltpu.sync_copy(idx_hbm.at[pl.ds(base + b * chunk, chunk)], iv)
            pltpu.async_copy(tab_hbm.at[iv], rv, sm)

        @pl.loop(0, nchunks - 2, step=2)
        def _(j):
            for b in range(2):
                iv, rv, sm = bufs[b]
                pltpu.make_async_copy(tab_hbm.at[iv], rv, sm).wait()
                pltpu.sync_copy(
                    rv, out_hbm.at[pl.ds(base + (j + b) * chunk, chunk)])
                pltpu.sync_copy(
                    idx_hbm.at[pl.ds(base + (j + b + 2) * chunk, chunk)], iv)
                pltpu.async_copy(tab_hbm.at[iv], rv, sm)

        for b in range(2):
            iv, rv, sm = bufs[b]
            pltpu.make_async_copy(tab_hbm.at[iv], rv, sm).wait()
            pltpu.sync_copy(
                rv,
                out_hbm.at[pl.ds(base + (nchunks - 2 + b) * chunk, chunk)])

    return k(table, idx)


def _sc_scatter_add(vals, idx, n_rows, chunk):
    e, dt = vals.shape
    epw = e // _NW
    cslab = min(dt, 64)
    nslabs = dt // cslab
    zeros = jnp.zeros((n_rows, cslab), vals.dtype)

    @functools.partial(
        pl.kernel,
        out_type=jax.ShapeDtypeStruct((2, n_rows, dt), vals.dtype),
        mesh=_sc_mesh(),
        scratch_types=[
            pltpu.VMEM((chunk,), jnp.int32),
            pltpu.VMEM((chunk, cslab), vals.dtype),
            pltpu.VMEM_SHARED((n_rows, cslab), vals.dtype),
        ],
        compiler_params=_SC_PARAMS,
    )
    def k(val_hbm, idx_hbm, z_hbm, out_hbm, idx_v, val_v, acc):
        cid = lax.axis_index("c")
        sid = lax.axis_index("s")
        base = (sid * 2 + cid) * epw

        for slab in range(nslabs):
            @pl.when(sid == 0)
            def _():
                pltpu.sync_copy(z_hbm, acc)

            plsc.subcore_barrier()

            @pl.loop(0, epw, step=chunk)
            def _(j):
                pltpu.sync_copy(idx_hbm.at[pl.ds(base + j, chunk)], idx_v)
                pltpu.sync_copy(
                    val_hbm.at[pl.ds(base + j, chunk),
                               pl.ds(slab * cslab, cslab)], val_v)
                pltpu.sync_copy(val_v, acc.at[idx_v], add=True)

            plsc.subcore_barrier()

            @pl.when(sid == 0)
            def _():
                pltpu.sync_copy(
                    acc, out_hbm.at[cid, :, pl.ds(slab * cslab, cslab)])

            if slab + 1 < nslabs:
                plsc.subcore_barrier()

    return k(vals, idx, zeros)


def _sc_scatter_add2(w, wt, idx, n_rows, chunk):
    e = w.shape[0]
    epw = e // _NW
    cslab = 64
    zw = jnp.zeros((n_rows, 16), w.dtype)
    zt = jnp.zeros((n_rows, cslab), wt.dtype)

    @functools.partial(
        pl.kernel,
        out_type=[
            jax.ShapeDtypeStruct((2, n_rows, 16), w.dtype),
            jax.ShapeDtypeStruct((2, n_rows, 128), wt.dtype),
        ],
        mesh=_sc_mesh(),
        scratch_types=[
            pltpu.VMEM((chunk,), jnp.int32),
            pltpu.VMEM((chunk, 16), jnp.float32),
            pltpu.VMEM((chunk, cslab), jnp.float32),
            pltpu.VMEM_SHARED((n_rows, 16), jnp.float32),
            pltpu.VMEM_SHARED((n_rows, cslab), jnp.float32),
        ],
        compiler_params=_SC_PARAMS,
    )
    def k(w_hbm, t_hbm, idx_hbm, zw_hbm, zt_hbm, ow_hbm, ot_hbm,
          idx_v, w_v, t_v, acc_w, acc_t):
        cid = lax.axis_index("c")
        sid = lax.axis_index("s")
        base = (sid * 2 + cid) * epw

        for slab in range(2):
            @pl.when(sid == 0)
            def _():
                pltpu.sync_copy(zt_hbm, acc_t)
                if slab == 0:
                    pltpu.sync_copy(zw_hbm, acc_w)

            plsc.subcore_barrier()

            @pl.loop(0, epw, step=chunk)
            def _(j):
                pltpu.sync_copy(idx_hbm.at[pl.ds(base + j, chunk)], idx_v)
                pltpu.sync_copy(
                    t_hbm.at[pl.ds(base + j, chunk),
                             pl.ds(slab * cslab, cslab)], t_v)
                pltpu.sync_copy(t_v, acc_t.at[idx_v], add=True)
                if slab == 0:
                    pltpu.sync_copy(
                        w_hbm.at[pl.ds(base + j, chunk), pl.ds(0, 16)], w_v)
                    pltpu.sync_copy(w_v, acc_w.at[idx_v], add=True)

            plsc.subcore_barrier()

            @pl.when(sid == 0)
            def _():
                pltpu.sync_copy(
                    acc_t, ot_hbm.at[cid, :, pl.ds(slab * cslab, cslab)])
                if slab == 0:
                    pltpu.sync_copy(acc_w, ow_hbm.at[cid, :, pl.ds(0, 16)])

            if slab == 0:
                plsc.subcore_barrier()

    return k(w, wt, idx, zw, zt)


def _tc_qkv(scalar, wcat, bcat):
    n, d = scalar.shape

    def body(x_ref, w_ref, b_ref, q_ref, kv_ref):
        y = jnp.dot(x_ref[...], w_ref[...], preferred_element_type=jnp.float32)
        y = y + b_ref[...]
        q_ref[...] = y[:, :d]
        kv_ref[...] = y[:, d:]

    return pl.pallas_call(
        body,
        out_shape=[
            jax.ShapeDtypeStruct((n, d), jnp.float32),
            jax.ShapeDtypeStruct((n, 2 * d), jnp.float32),
        ],
    )(scalar, wcat, bcat.reshape(1, 3 * d))


def _tc_attn_w(qr, kvc, el, em1, emb1, em2p, em2b, blk):
    e = qr.shape[0]

    def body(q_ref, k_ref, l_ref, w1_ref, b1_ref, w2_ref, b2_ref, o_ref):
        d_i = lax.broadcasted_iota(jnp.int32, (_D, 16), 0) // _DH
        h_i = lax.broadcasted_iota(jnp.int32, (_D, 16), 1)
        mask = (d_i == h_i).astype(jnp.float32)
        dots = jnp.dot(q_ref[...] * k_ref[...], mask,
                       preferred_element_type=jnp.float32) * 0.25
        l = l_ref[...]
        t = l * w1_ref[...] + b1_ref[...]
        t = t * jax.nn.sigmoid(t)
        bias = jnp.dot(t, w2_ref[...], preferred_element_type=jnp.float32)
        bias = bias + b2_ref[...]
        cut = 0.5 * (jnp.cos(l * (math.pi / _CUTOFF)) + 1.0)
        cut = cut * (l < _CUTOFF).astype(jnp.float32)
        a = (dots + bias) * cut
        hm = (lax.broadcasted_iota(jnp.int32, (1, 16), 1) < _H)
        o_ref[...] = jnp.exp(a) * hm.astype(jnp.float32)

    return pl.pallas_call(
        body,
        grid=(e // blk,),
        in_specs=[
            pl.BlockSpec((blk, _D), lambda i: (i, 0)),
            pl.BlockSpec((blk, _D), lambda i: (i, 0)),
            pl.BlockSpec((blk, 1), lambda i: (i, 0)),
            pl.BlockSpec((1, _D), lambda i: (0, 0)),
            pl.BlockSpec((1, _D), lambda i: (0, 0)),
            pl.BlockSpec((_D, 16), lambda i: (0, 0)),
            pl.BlockSpec((1, 16), lambda i: (0, 0)),
        ],
        out_shape=jax.ShapeDtypeStruct((e, 16), jnp.float32),
        out_specs=pl.BlockSpec((blk, 16), lambda i: (i, 0)),
    )(qr, kvc, el, em1, emb1, em2p, em2b)


def _tc_wv(w, kvc, blk):
    e = w.shape[0]

    def body(w_ref, v_ref, o_ref):
        h_i = lax.broadcasted_iota(jnp.int32, (16, _D), 0)
        d_i = lax.broadcasted_iota(jnp.int32, (16, _D), 1) // _DH
        ex = (h_i == d_i).astype(jnp.float32)
        o_ref[...] = jnp.dot(w_ref[...], ex,
                             preferred_element_type=jnp.float32) * v_ref[...]

    return pl.pallas_call(
        body,
        grid=(e // blk,),
        in_specs=[
            pl.BlockSpec((blk, 16), lambda i: (i, 0)),
            pl.BlockSpec((blk, _D), lambda i: (i, 1)),
        ],
        out_shape=jax.ShapeDtypeStruct((e, _D), jnp.float32),
        out_specs=pl.BlockSpec((blk, _D), lambda i: (i, 0)),
    )(w, kvc)


def _tc_post_attn(pw, pt, scalar, owt, ob, lng, lnb):
    n, d = scalar.shape

    def body(pw_ref, pt_ref, x_ref, w_ref, b_ref, g_ref, bb_ref, o_ref):
        sw = pw_ref[0] + pw_ref[1]
        st = pt_ref[0] + pt_ref[1]
        h_i = lax.broadcasted_iota(jnp.int32, (16, _D), 0)
        d_i = lax.broadcasted_iota(jnp.int32, (16, _D), 1) // _DH
        ex = (h_i == d_i).astype(jnp.float32)
        den = jnp.dot(sw, ex, preferred_element_type=jnp.float32) + 1e-8
        o = st / den
        o = jnp.dot(o, w_ref[...], preferred_element_type=jnp.float32)
        o = o + b_ref[...]
        h = x_ref[...] + o
        m = jnp.mean(h, axis=-1, keepdims=True)
        v = jnp.mean((h - m) ** 2, axis=-1, keepdims=True)
        o_ref[...] = (h - m) / jnp.sqrt(v + 1e-5) * g_ref[...] + bb_ref[...]

    return pl.pallas_call(
        body, out_shape=jax.ShapeDtypeStruct((n, d), jnp.float32)
    )(pw, pt, scalar, owt, ob.reshape(1, d), lng.reshape(1, d),
      lnb.reshape(1, d))


def _tc_sw(el, sm1, smb1, sm2t, smb2, blk):
    e = el.shape[0]

    def body(l_ref, w1_ref, b1_ref, w2_ref, b2_ref, o_ref):
        t = l_ref[...] * w1_ref[...] + b1_ref[...]
        t = t * jax.nn.sigmoid(t)
        o_ref[...] = jnp.dot(t, w2_ref[...],
                             preferred_element_type=jnp.float32) + b2_ref[...]

    return pl.pallas_call(
        body,
        grid=(e // blk,),
        in_specs=[
            pl.BlockSpec((blk, 1), lambda i: (i, 0)),
            pl.BlockSpec((1, _D), lambda i: (0, 0)),
            pl.BlockSpec((1, _D), lambda i: (0, 0)),
            pl.BlockSpec((_D, _D), lambda i: (0, 0)),
            pl.BlockSpec((1, _D), lambda i: (0, 0)),
        ],
        out_shape=jax.ShapeDtypeStruct((e, _D), jnp.float32),
        out_specs=pl.BlockSpec((blk, _D), lambda i: (i, 0)),
    )(el, sm1, smb1, sm2t, smb2)


def _tc_mul(a, b, blk):
    e, d = a.shape

    def body(a_ref, b_ref, o_ref):
        o_ref[...] = a_ref[...] * b_ref[...]

    return pl.pallas_call(
        body,
        grid=(e // blk,),
        in_specs=[
            pl.BlockSpec((blk, d), lambda i: (i, 0)),
            pl.BlockSpec((blk, d), lambda i: (i, 0)),
        ],
        out_shape=jax.ShapeDtypeStruct((e, d), jnp.float32),
        out_specs=pl.BlockSpec((blk, d), lambda i: (i, 0)),
    )(a, b)


def _tc_final(scalar, tp_p, fw1t, fb1, fw2t, fb2, fng, fnb, blk):
    n, d = scalar.shape
    dh = fw1t.shape[1]

    def body(x_ref, t_ref, w1_ref, b1_ref, w2_ref, b2_ref, g_ref, bb_ref,
             o_ref):
        so = x_ref[...] + t_ref[0] + t_ref[1]
        m = jnp.mean(so, axis=-1, keepdims=True)
        v = jnp.mean((so - m) ** 2, axis=-1, keepdims=True)
        xn = (so - m) / jnp.sqrt(v + 1e-5) * g_ref[...] + bb_ref[...]
        hdn = jnp.dot(xn, w1_ref[...], preferred_element_type=jnp.float32)
        hdn = hdn + b1_ref[...]
        hdn = 0.5 * hdn * (1.0 + lax.erf(hdn * (1.0 / math.sqrt(2.0))))
        o_ref[...] = so + jnp.dot(hdn, w2_ref[...],
                                  preferred_element_type=jnp.float32) + b2_ref[...]

    return pl.pallas_call(
        body,
        grid=(n // blk,),
        in_specs=[
            pl.BlockSpec((blk, d), lambda i: (i, 0)),
            pl.BlockSpec((2, blk, d), lambda i: (0, i, 0)),
            pl.BlockSpec((d, dh), lambda i: (0, 0)),
            pl.BlockSpec((1, dh), lambda i: (0, 0)),
            pl.BlockSpec((dh, d), lambda i: (0, 0)),
            pl.BlockSpec((1, d), lambda i: (0, 0)),
            pl.BlockSpec((1, d), lambda i: (0, 0)),
            pl.BlockSpec((1, d), lambda i: (0, 0)),
        ],
        out_shape=jax.ShapeDtypeStruct((n, d), jnp.float32),
        out_specs=pl.BlockSpec((blk, d), lambda i: (i, 0)),
    )(scalar, tp_p, fw1t, fb1.reshape(1, dh), fw2t, fb2.reshape(1, d),
      fng.reshape(1, d), fnb.reshape(1, d))


def kernel(scalar, vector, edge_index, edge_vec, edge_length, edge_sh,
           qW, qb, kW, kb, vW, vb, emW1, emb1, emW2, emb2, oW, ob,
           lng, lnb, smW1, smb1, smW2, smb2, fW1, fb1, fW2, fb2, fng, fnb):
    row = edge_index[0]
    col = edge_index[1]
    n, d = scalar.shape

    wcat = jnp.concatenate([qW.T, kW.T, vW.T], axis=1)
    bcat = jnp.concatenate([qb, kb, vb])
    q, kv = _tc_qkv(scalar, wcat, bcat)

    sw = _tc_sw(edge_length, smW1.reshape(1, d), smb1.reshape(1, d),
                smW2.T, smb2.reshape(1, d), 4000)

    qr = _sc_gather(q, row, 200)
    kvc = _sc_gather(kv, col, 40)

    em2p = jnp.zeros((d, 16), jnp.float32).at[:, :_H].set(emW2.T)
    em2b = jnp.zeros((1, 16), jnp.float32).at[:, :_H].set(emb2)
    w = _tc_attn_w(qr, kvc, edge_length, emW1.reshape(1, d),
                   emb1.reshape(1, d), em2p, em2b, 4000)

    weighted = _tc_wv(w, kvc, 4000)
    pw, pt = _sc_scatter_add2(w, weighted, row, n, 400)
    h_attn = _tc_post_attn(pw, pt, scalar, oW.T, ob, lng, lnb)

    hc = _sc_gather(h_attn, col, 200)
    prod = _tc_mul(hc, sw, 4000)
    tp_p = _sc_scatter_add(prod, row, n, 400)

    out = _tc_final(scalar, tp_p, fW1.T, fb1, fW2.T, fb2, fng, fnb, 2000)
    return (out, vector)
